# baseline (speedup 1.0000x reference)
import functools

import jax
import jax.numpy as jnp
from jax import lax
from jax.experimental import pallas as pl
from jax.experimental.pallas import tpu as pltpu
from jax.experimental.pallas import tpu_sc as plsc

D = 64
FAN = 10
CHUNK = 128


def _sc_gather(table, idx2, idx1, idx0, w1, n2, n1, n0):
    info = plsc.get_sparse_core_info()
    nc, ns = info.num_cores, info.num_subcores
    ch2, ch1 = idx2.shape[1], idx1.shape[1]
    ch0 = idx0.shape[1] if idx0 is not None else 0

    mesh = plsc.VectorSubcoreMesh(core_axis_name="c", subcore_axis_name="s")

    out_type = [
        jax.ShapeDtypeStruct((n2, 128), jnp.float32),
        jax.ShapeDtypeStruct((n1, 128), jnp.float32),
    ]
    scratch = [
        pltpu.VMEM((ch2, CHUNK), jnp.int32),
        pltpu.VMEM((ch1, CHUNK), jnp.int32),
        pltpu.VMEM((ch2, CHUNK), jnp.float32),
        pltpu.VMEM((2, CHUNK, 128), jnp.float32),
        pltpu.SemaphoreType.DMA,
    ]
    if idx0 is not None:
        out_type.append(jax.ShapeDtypeStruct((n0, 128), jnp.float32))
        scratch.insert(2, pltpu.VMEM((ch0, CHUNK), jnp.int32))

    @functools.partial(
        pl.kernel,
        mesh=mesh,
        compiler_params=pltpu.CompilerParams(use_tc_tiling_on_sc=False,
                                             needs_layout_passes=False),
        out_type=tuple(out_type),
        scratch_types=scratch,
    )
    def gather_kernel(table_hbm, i2_hbm, i1_hbm, *rest):
        if idx0 is not None:
            (i0_hbm, w1_hbm, e2_hbm, e1_hbm, e0_hbm,
             i2_v, i1_v, i0_v, w1_v, rows_v, sem) = rest
        else:
            (w1_hbm, e2_hbm, e1_hbm,
             i2_v, i1_v, w1_v, rows_v, sem) = rest
        wid = lax.axis_index("s") * nc + lax.axis_index("c")

        def run(idx_hbm, idx_v, nch, out_hbm, scribble_w):
            pltpu.sync_copy(idx_hbm.at[wid], idx_v)
            base = wid * nch * CHUNK
            pltpu.async_copy(table_hbm.at[idx_v.at[0]], rows_v.at[0], sem)

            def body(i, carry):
                slot = lax.rem(i, 2)
                nslot = lax.rem(i + 1, 2)

                @pl.when(i + 1 < nch)
                def _():
                    pltpu.async_copy(table_hbm.at[idx_v.at[i + 1]],
                                     rows_v.at[nslot], sem)

                pltpu.make_async_copy(table_hbm.at[idx_v.at[i]],
                                      rows_v.at[slot], sem).wait()
                if scribble_w:
                    def grpfn(g, c):
                        w16 = w1_v[i, pl.ds(16 * g, 16)]
                        idx_r = 16 * g + lax.broadcasted_iota(
                            jnp.int32, (16,), 0)
                        idx_c = jnp.full((16,), D, jnp.int32)
                        plsc.store_scatter(rows_v.at[slot],
                                           [idx_r, idx_c], w16)
                        return c

                    lax.fori_loop(0, CHUNK // 16, grpfn, 0)
                pltpu.sync_copy(rows_v.at[slot],
                                out_hbm.at[pl.ds(base + i * CHUNK, CHUNK)])
                return carry

            lax.fori_loop(0, nch, body, 0)

        pltpu.sync_copy(w1_hbm.at[wid], w1_v)
        run(i2_hbm, i2_v, ch2, e2_hbm, True)
        run(i1_hbm, i1_v, ch1, e1_hbm, False)
        if idx0 is not None:
            run(i0_hbm, i0_v, ch0, e0_hbm, False)

    if idx0 is not None:
        return gather_kernel(table, idx2, idx1, idx0, w1)
    return gather_kernel(table, idx2, idx1, w1)


def _l2n(z):
    zn = jnp.sqrt(jnp.sum(z * z, axis=1, keepdims=True))
    zn = jnp.where(zn == 0, jnp.float32(1.0), zn)
    return z / zn


def _dense_body(BI, PARTS):
    def body(e0, e1, e2, w0, Fbig, Fsml, WB0t, Wq1t, WA1t, WB1t, WG1t,
             WG2t, av, c1v, bq1v, bw1v, bG1v, out):
        a = av[...]
        c1 = c1v[...]
        Fb = Fbig[...]
        Fs = Fsml[...]

        def fansum_big(t):
            rp = t.shape[0] // PARTS
            return jnp.concatenate(
                [jnp.dot(Fb, t[p * rp:(p + 1) * rp]) for p in range(PARTS)],
                axis=0)

        x2 = e2[...]
        t2 = jnp.maximum(x2[:, :D] + a, 0.0) * x2[:, D:D + 1]
        wn1 = fansum_big(t2)
        x1 = e1[...]
        h1 = _l2n(jnp.maximum(
            x1[:, D:] + c1 + jnp.dot(wn1, WB0t[...]), 0.0))
        t1 = jnp.maximum(x1[:, :D] + a, 0.0) * w0[...]
        wn0 = jnp.dot(Fs, t1)
        h0 = _l2n(jnp.maximum(
            e0[...][:, D:] + c1 + jnp.dot(wn0, WB0t[...]), 0.0))
        nb = jnp.maximum(jnp.dot(h1, Wq1t[...]) + bq1v[...], 0.0) * w0[...]
        wn = jnp.dot(Fs, nb)
        hF = _l2n(jnp.maximum(
            jnp.dot(h0, WA1t[...]) + jnp.dot(wn, WB1t[...]) + bw1v[...], 0.0))
        g = jnp.maximum(jnp.dot(hF, WG1t[...]) + bG1v[...], 0.0)
        out[...] = jnp.dot(g, WG2t[...])

    return body


def _dense(E0, E1, E2, w0c, WB0t, Wq1t, WA1t, WB1t, WG1t, WG2t,
           a, c1, bq1, bw1, bG1, B):
    BI = 64
    PARTS = 8
    grid = (B // BI,)

    nbig = BI * FAN * FAN // PARTS
    Fbig = jnp.kron(jnp.eye(nbig // FAN, dtype=jnp.float32),
                    jnp.ones((1, FAN), jnp.float32))
    Fsml = jnp.kron(jnp.eye(BI, dtype=jnp.float32),
                    jnp.ones((1, FAN), jnp.float32))

    def full(shape):
        return pl.BlockSpec(shape, lambda i: (0,) * len(shape))

    w_spec = full((D, D))
    b_spec = full((1, D))
    return pl.pallas_call(
        _dense_body(BI, PARTS),
        grid=grid,
        in_specs=[
            pl.BlockSpec((BI, 2 * D), lambda i: (i, 0)),
            pl.BlockSpec((BI * FAN, 2 * D), lambda i: (i, 0)),
            pl.BlockSpec((BI * FAN * FAN, 2 * D), lambda i: (i, 0)),
            pl.BlockSpec((BI * FAN, 1), lambda i: (i, 0)),
            full((nbig // FAN, nbig)), full((BI, BI * FAN)),
            w_spec, w_spec, w_spec, w_spec, w_spec, w_spec,
            b_spec, b_spec, b_spec, b_spec, b_spec,
        ],
        out_specs=pl.BlockSpec((BI, D), lambda i: (i, 0)),
        out_shape=jax.ShapeDtypeStruct((B, D), jnp.float32),
    )(E0, E1, E2, w0c, Fbig, Fsml, WB0t, Wq1t, WA1t, WB1t, WG1t, WG2t,
      a, c1, bq1, bw1, bG1)


def kernel(items, neighbors0, neighbors1, weights0, weights1, offsets0,
           offsets1, item_table, Wp, bp, Wq0, bq0, Ww0, bw0, Wq1, bq1, Ww1,
           bw1, WG1, bG1, WG2):
    B = items.shape[0]
    n1 = neighbors0.shape[0]
    n2 = neighbors1.shape[0]

    info = plsc.get_sparse_core_info()
    nw = info.num_cores * info.num_subcores

    WwA0, WwB0 = Ww0[:, :D], Ww0[:, D:]
    WwA1, WwB1 = Ww1[:, :D], Ww1[:, D:]
    A = Wq0 @ Wp
    a = (Wq0 @ bp + bq0)[None, :]
    C1 = WwA0 @ Wp
    c1 = (WwA0 @ bp + bw0)[None, :]

    W2 = jnp.concatenate([A.T, C1.T], axis=1)
    TAC = item_table @ W2

    com = (WwB0.T, Wq1.T, WwA1.T, WwB1.T, WG1.T, WG2.T,
           a, c1, bq1[None, :], bw1[None, :], bG1[None, :])

    S = 4
    Bs, n1s, n2s = B // S, n1 // S, n2 // S
    idx0 = items.astype(jnp.int32).reshape(nw, -1, CHUNK)
    E2 = [None] * S
    E2[0], E1, E0 = _sc_gather(
        TAC,
        neighbors1[:n2s].astype(jnp.int32).reshape(nw, -1, CHUNK),
        neighbors0.astype(jnp.int32).reshape(nw, -1, CHUNK),
        idx0,
        weights1[:n2s].reshape(nw, -1, CHUNK), n2s, n1, B)
    for q in range(1, S):
        E2[q], _ = _sc_gather(
            TAC,
            neighbors1[q * n2s:(q + 1) * n2s].astype(jnp.int32)
            .reshape(nw, -1, CHUNK),
            neighbors0[:128 * nw].astype(jnp.int32).reshape(nw, 1, CHUNK),
            None,
            weights1[q * n2s:(q + 1) * n2s].reshape(nw, -1, CHUNK),
            n2s, 128 * nw, 0)
    outs = []
    for q in range(S):
        outs.append(_dense(
            E0[q * Bs:(q + 1) * Bs], E1[q * n1s:(q + 1) * n1s], E2[q],
            weights0[q * n1s:(q + 1) * n1s, None], *com, B=Bs))
    return jnp.concatenate(outs, axis=0)


# speedup vs baseline: 1.0417x; 1.0417x over previous
import functools

import jax
import jax.numpy as jnp
from jax import lax
from jax.experimental import pallas as pl
from jax.experimental.pallas import tpu as pltpu
from jax.experimental.pallas import tpu_sc as plsc

D = 64
FAN = 10
CHUNK = 128


def _sc_gather(table, idx2, idx1, idx0, w1, n2, n1, n0):
    info = plsc.get_sparse_core_info()
    nc, ns = info.num_cores, info.num_subcores
    ch2, ch1 = idx2.shape[1], idx1.shape[1]
    ch0 = idx0.shape[1] if idx0 is not None else 0

    mesh = plsc.VectorSubcoreMesh(core_axis_name="c", subcore_axis_name="s")

    out_type = [
        jax.ShapeDtypeStruct((n2, 128), jnp.float32),
        jax.ShapeDtypeStruct((n1, 128), jnp.float32),
    ]
    scratch = [
        pltpu.VMEM((ch2, CHUNK), jnp.int32),
        pltpu.VMEM((ch1, CHUNK), jnp.int32),
        pltpu.VMEM((ch2, CHUNK), jnp.float32),
        pltpu.VMEM((2, CHUNK, 128), jnp.float32),
        pltpu.SemaphoreType.DMA,
    ]
    if idx0 is not None:
        out_type.append(jax.ShapeDtypeStruct((n0, 128), jnp.float32))
        scratch.insert(2, pltpu.VMEM((ch0, CHUNK), jnp.int32))

    @functools.partial(
        pl.kernel,
        mesh=mesh,
        compiler_params=pltpu.CompilerParams(use_tc_tiling_on_sc=False,
                                             needs_layout_passes=False),
        out_type=tuple(out_type),
        scratch_types=scratch,
    )
    def gather_kernel(table_hbm, i2_hbm, i1_hbm, *rest):
        if idx0 is not None:
            (i0_hbm, w1_hbm, e2_hbm, e1_hbm, e0_hbm,
             i2_v, i1_v, i0_v, w1_v, rows_v, sem) = rest
        else:
            (w1_hbm, e2_hbm, e1_hbm,
             i2_v, i1_v, w1_v, rows_v, sem) = rest
        wid = lax.axis_index("s") * nc + lax.axis_index("c")

        def run(idx_hbm, idx_v, nch, out_hbm, scribble_w):
            pltpu.sync_copy(idx_hbm.at[wid], idx_v)
            base = wid * nch * CHUNK
            pltpu.async_copy(table_hbm.at[idx_v.at[0]], rows_v.at[0], sem)

            def body(i, carry):
                slot = lax.rem(i, 2)
                nslot = lax.rem(i + 1, 2)

                @pl.when(i + 1 < nch)
                def _():
                    pltpu.async_copy(table_hbm.at[idx_v.at[i + 1]],
                                     rows_v.at[nslot], sem)

                pltpu.make_async_copy(table_hbm.at[idx_v.at[i]],
                                      rows_v.at[slot], sem).wait()
                if scribble_w:
                    def grpfn(g, c):
                        w16 = w1_v[i, pl.ds(16 * g, 16)]
                        idx_r = 16 * g + lax.broadcasted_iota(
                            jnp.int32, (16,), 0)
                        idx_c = jnp.full((16,), D, jnp.int32)
                        plsc.store_scatter(rows_v.at[slot],
                                           [idx_r, idx_c], w16)
                        return c

                    lax.fori_loop(0, CHUNK // 16, grpfn, 0)
                pltpu.sync_copy(rows_v.at[slot],
                                out_hbm.at[pl.ds(base + i * CHUNK, CHUNK)])
                return carry

            lax.fori_loop(0, nch, body, 0)

        pltpu.sync_copy(w1_hbm.at[wid], w1_v)
        run(i2_hbm, i2_v, ch2, e2_hbm, True)
        run(i1_hbm, i1_v, ch1, e1_hbm, False)
        if idx0 is not None:
            run(i0_hbm, i0_v, ch0, e0_hbm, False)

    if idx0 is not None:
        return gather_kernel(table, idx2, idx1, idx0, w1)
    return gather_kernel(table, idx2, idx1, w1)


def _l2n(z):
    zn = jnp.sqrt(jnp.sum(z * z, axis=1, keepdims=True))
    zn = jnp.where(zn == 0, jnp.float32(1.0), zn)
    return z / zn


def _dense_body(BI, PARTS):
    def body(e0, e1, e2, w0, Fbig, Fsml, WB0t, Wq1t, WA1t, WB1t, WG1t,
             WG2t, av, c1v, bq1v, bw1v, bG1v, out):
        a = av[...]
        c1 = c1v[...]
        Fb = Fbig[...]
        Fs = Fsml[...]

        def fansum_big(t):
            rp = t.shape[0] // PARTS
            return jnp.concatenate(
                [jnp.dot(Fb, t[p * rp:(p + 1) * rp]) for p in range(PARTS)],
                axis=0)

        x2 = e2[...]
        t2 = jnp.maximum(x2[:, :D] + a, 0.0) * x2[:, D:D + 1]
        wn1 = fansum_big(t2)
        x1 = e1[...]
        h1 = _l2n(jnp.maximum(
            x1[:, D:] + c1 + jnp.dot(wn1, WB0t[...]), 0.0))
        t1 = jnp.maximum(x1[:, :D] + a, 0.0) * w0[...]
        wn0 = jnp.dot(Fs, t1)
        h0 = _l2n(jnp.maximum(
            e0[...][:, D:] + c1 + jnp.dot(wn0, WB0t[...]), 0.0))
        nb = jnp.maximum(jnp.dot(h1, Wq1t[...]) + bq1v[...], 0.0) * w0[...]
        wn = jnp.dot(Fs, nb)
        hF = _l2n(jnp.maximum(
            jnp.dot(h0, WA1t[...]) + jnp.dot(wn, WB1t[...]) + bw1v[...], 0.0))
        g = jnp.maximum(jnp.dot(hF, WG1t[...]) + bG1v[...], 0.0)
        out[...] = jnp.dot(g, WG2t[...])

    return body


def _dense(E0, E1, E2, w0c, WB0t, Wq1t, WA1t, WB1t, WG1t, WG2t,
           a, c1, bq1, bw1, bG1, B):
    BI = 64
    PARTS = 8
    grid = (B // BI,)

    nbig = BI * FAN * FAN // PARTS
    Fbig = jnp.kron(jnp.eye(nbig // FAN, dtype=jnp.float32),
                    jnp.ones((1, FAN), jnp.float32))
    Fsml = jnp.kron(jnp.eye(BI, dtype=jnp.float32),
                    jnp.ones((1, FAN), jnp.float32))

    def full(shape):
        return pl.BlockSpec(shape, lambda i: (0,) * len(shape))

    w_spec = full((D, D))
    b_spec = full((1, D))
    return pl.pallas_call(
        _dense_body(BI, PARTS),
        grid=grid,
        in_specs=[
            pl.BlockSpec((BI, 2 * D), lambda i: (i, 0)),
            pl.BlockSpec((BI * FAN, 2 * D), lambda i: (i, 0)),
            pl.BlockSpec((BI * FAN * FAN, 2 * D), lambda i: (i, 0)),
            pl.BlockSpec((BI * FAN, 1), lambda i: (i, 0)),
            full((nbig // FAN, nbig)), full((BI, BI * FAN)),
            w_spec, w_spec, w_spec, w_spec, w_spec, w_spec,
            b_spec, b_spec, b_spec, b_spec, b_spec,
        ],
        out_specs=pl.BlockSpec((BI, D), lambda i: (i, 0)),
        out_shape=jax.ShapeDtypeStruct((B, D), jnp.float32),
    )(E0, E1, E2, w0c, Fbig, Fsml, WB0t, Wq1t, WA1t, WB1t, WG1t, WG2t,
      a, c1, bq1, bw1, bG1)


def kernel(items, neighbors0, neighbors1, weights0, weights1, offsets0,
           offsets1, item_table, Wp, bp, Wq0, bq0, Ww0, bw0, Wq1, bq1, Ww1,
           bw1, WG1, bG1, WG2):
    B = items.shape[0]
    n1 = neighbors0.shape[0]
    n2 = neighbors1.shape[0]

    info = plsc.get_sparse_core_info()
    nw = info.num_cores * info.num_subcores

    WwA0, WwB0 = Ww0[:, :D], Ww0[:, D:]
    WwA1, WwB1 = Ww1[:, :D], Ww1[:, D:]
    A = Wq0 @ Wp
    a = (Wq0 @ bp + bq0)[None, :]
    C1 = WwA0 @ Wp
    c1 = (WwA0 @ bp + bw0)[None, :]

    W2 = jnp.concatenate([A.T, C1.T], axis=1)
    TAC = item_table @ W2

    com = (WwB0.T, Wq1.T, WwA1.T, WwB1.T, WG1.T, WG2.T,
           a, c1, bq1[None, :], bw1[None, :], bG1[None, :])

    Bh, n1h, n2h = B // 2, n1 // 2, n2 // 2
    idx0 = items.astype(jnp.int32).reshape(nw, -1, CHUNK)
    E2a, E1a, E0 = _sc_gather(
        TAC,
        neighbors1[:n2h].astype(jnp.int32).reshape(nw, -1, CHUNK),
        neighbors0[:n1h].astype(jnp.int32).reshape(nw, -1, CHUNK),
        idx0,
        weights1[:n2h].reshape(nw, -1, CHUNK), n2h, n1h, B)
    E2b, E1b = _sc_gather(
        TAC,
        neighbors1[n2h:].astype(jnp.int32).reshape(nw, -1, CHUNK),
        neighbors0[n1h:].astype(jnp.int32).reshape(nw, -1, CHUNK),
        None,
        weights1[n2h:].reshape(nw, -1, CHUNK), n2h, n1h, 0)
    outa = _dense(E0[:Bh], E1a, E2a, weights0[:n1h, None], *com, B=Bh)
    outb = _dense(E0[Bh:], E1b, E2b, weights0[n1h:, None], *com, B=Bh)
    return jnp.concatenate([outa, outb], axis=0)
